# baseline (speedup 1.0000x reference)
import functools

import jax
import jax.numpy as jnp
from jax import lax
from jax.experimental import pallas as pl
from jax.experimental.pallas import tpu as pltpu
from jax.experimental.pallas import tpu_sc as plsc

_CHUNK = 16
_NBUF = 6


def _make_sc_copy(rows, d_model, dtype):
    info = plsc.get_sparse_core_info()
    nc, ns = info.num_cores, info.num_subcores
    nw = nc * ns
    assert rows % nw == 0
    rows_per_w = rows // nw
    chunk = min(_CHUNK, rows_per_w)
    assert rows_per_w % chunk == 0
    nch = rows_per_w // chunk
    nbuf = min(_NBUF, nch)

    mesh = plsc.VectorSubcoreMesh(core_axis_name="c", subcore_axis_name="s")

    @functools.partial(
        pl.kernel,
        mesh=mesh,
        out_type=jax.ShapeDtypeStruct((rows, d_model), dtype),
        scratch_types=[
            pltpu.VMEM((nbuf * chunk, d_model), dtype),
            pltpu.SemaphoreType.DMA,
            pltpu.SemaphoreType.DMA,
        ],
    )
    def copy_k(w_hbm, out_hbm, buf, gsem, ssem):
        wid = lax.axis_index("s") * nc + lax.axis_index("c")
        base = wid * rows_per_w

        def slot(i):
            return buf.at[pl.ds((i % nbuf) * chunk, chunk)]

        def gather(i):
            return pltpu.make_async_copy(
                w_hbm.at[pl.ds(base + i * chunk, chunk)], slot(i), gsem)

        def scatter(i):
            return pltpu.make_async_copy(
                slot(i), out_hbm.at[pl.ds(base + i * chunk, chunk)], ssem)

        for j in range(nbuf - 1):
            gather(j).start()
        for i in range(nch):
            gather(i).wait()
            nxt = i + nbuf - 1
            if nxt < nch:
                if nxt >= nbuf:
                    scatter(nxt - nbuf).wait()
                gather(nxt).start()
            scatter(i).start()
        for i in range(max(0, nch - nbuf), nch):
            scatter(i).wait()

    return copy_k


def kernel(x, pos_emb_weight):
    seq_len = x.shape[1]
    rows = pos_emb_weight[:seq_len]
    out = _make_sc_copy(rows.shape[0], rows.shape[1], rows.dtype)(rows)
    return out[None]


# speedup vs baseline: 1.0002x; 1.0002x over previous
import functools

import jax
import jax.numpy as jnp
from jax import lax
from jax.experimental import pallas as pl
from jax.experimental.pallas import tpu as pltpu
from jax.experimental.pallas import tpu_sc as plsc

_CHUNK = 16
_NBUF = 6


def _make_sc_copy(rows, d_model, dtype):
    info = plsc.get_sparse_core_info()
    nc, ns = info.num_cores, info.num_subcores
    nw = nc * ns
    assert rows % nw == 0
    rows_per_w = rows // nw
    chunk = min(_CHUNK, rows_per_w)
    assert rows_per_w % chunk == 0
    nch = rows_per_w // chunk
    nbuf = min(_NBUF, nch)

    mesh = plsc.VectorSubcoreMesh(core_axis_name="c", subcore_axis_name="s")

    @functools.partial(
        pl.kernel,
        mesh=mesh,
        out_type=jax.ShapeDtypeStruct((rows, d_model), dtype),
        scratch_types=(
            [pltpu.VMEM((chunk, d_model), dtype) for _ in range(nbuf)]
            + [pltpu.SemaphoreType.DMA, pltpu.SemaphoreType.DMA]
        ),
    )
    def copy_k(w_hbm, out_hbm, *scratch):
        bufs = scratch[:nbuf]
        gsem, ssem = scratch[nbuf], scratch[nbuf + 1]
        wid = lax.axis_index("s") * nc + lax.axis_index("c")
        base = wid * rows_per_w

        def gather(i):
            return pltpu.make_async_copy(
                w_hbm.at[pl.ds(base + i * chunk, chunk)], bufs[i % nbuf], gsem)

        def scatter(i):
            return pltpu.make_async_copy(
                bufs[i % nbuf], out_hbm.at[pl.ds(base + i * chunk, chunk)], ssem)

        for j in range(nbuf - 1):
            gather(j).start()
        for i in range(nch):
            gather(i).wait()
            nxt = i + nbuf - 1
            if nxt < nch:
                if nxt >= nbuf:
                    scatter(nxt - nbuf).wait()
                gather(nxt).start()
            scatter(i).start()
        for i in range(max(0, nch - nbuf), nch):
            scatter(i).wait()

    return copy_k


def kernel(x, pos_emb_weight):
    seq_len = x.shape[1]
    rows = pos_emb_weight[:seq_len]
    out = _make_sc_copy(rows.shape[0], rows.shape[1], rows.dtype)(rows)
    return out[None]


# speedup vs baseline: 1.0390x; 1.0388x over previous
import functools

import jax
import jax.numpy as jnp
from jax import lax
from jax.experimental import pallas as pl
from jax.experimental.pallas import tpu as pltpu
from jax.experimental.pallas import tpu_sc as plsc

_CHUNK = 16
_NBUF = 6


def _make_sc_copy(rows, d_model, dtype):
    info = plsc.get_sparse_core_info()
    nc, ns = info.num_cores, info.num_subcores
    nw = nc * ns
    assert rows % nw == 0
    rows_per_w = rows // nw
    chunk = min(_CHUNK, rows_per_w)
    assert rows_per_w % chunk == 0
    nch = rows_per_w // chunk
    nbuf = min(_NBUF, nch)

    mesh = plsc.VectorSubcoreMesh(core_axis_name="c", subcore_axis_name="s")

    @functools.partial(
        pl.kernel,
        mesh=mesh,
        out_type=jax.ShapeDtypeStruct((rows, d_model), dtype),
        scratch_types=(
            [pltpu.VMEM((chunk, d_model), dtype) for _ in range(nbuf)]
            + [pltpu.SemaphoreType.DMA for _ in range(2 * nbuf)]
        ),
    )
    def copy_k(w_hbm, out_hbm, *scratch):
        bufs = scratch[:nbuf]
        gsems = scratch[nbuf:2 * nbuf]
        ssems = scratch[2 * nbuf:]
        wid = lax.axis_index("s") * nc + lax.axis_index("c")
        base = wid * rows_per_w

        def gather(i):
            return pltpu.make_async_copy(
                w_hbm.at[pl.ds(base + i * chunk, chunk)],
                bufs[i % nbuf], gsems[i % nbuf])

        def scatter(i):
            return pltpu.make_async_copy(
                bufs[i % nbuf],
                out_hbm.at[pl.ds(base + i * chunk, chunk)], ssems[i % nbuf])

        for j in range(nbuf - 1):
            gather(j).start()
        for i in range(nch):
            gather(i).wait()
            nxt = i + nbuf - 1
            if nxt < nch:
                if nxt >= nbuf:
                    scatter(nxt - nbuf).wait()
                gather(nxt).start()
            scatter(i).start()
        for i in range(max(0, nch - nbuf), nch):
            scatter(i).wait()

    return copy_k


def kernel(x, pos_emb_weight):
    seq_len = x.shape[1]
    rows = pos_emb_weight[:seq_len]
    out = _make_sc_copy(rows.shape[0], rows.shape[1], rows.dtype)(rows)
    return out[None]
